# baseline (speedup 1.0000x reference)
import jax
import jax.numpy as jnp
from jax import lax
from jax.experimental import pallas as pl
from jax.experimental.pallas import tpu as pltpu
from jax.experimental.pallas import tpu_sc as plsc

_B, _C, _H, _W, _K, _NC = 8, 2, 128, 128, 128, 80
_HW = _H * _W
_NSLOT = _B * _K
_NCORE = 2
_NSUB = 16
_NW = _NCORE * _NSUB
_PER = _NSLOT // _NW
_NCHUNK = _PER // 16


def _smooth_l1_vec(d):
    a = jnp.abs(d)
    return jnp.where(a < 1.0, 0.5 * a * a, a - 0.5)


def _sc_loss_kernel(out_flat, hm_flat, ind, inde_flat, mask, tgt_flat,
                    o_parts,
                    ind_v, inde_v, mask_v, tgt_v,
                    idx0_v, idx1_v, idxh_v,
                    pred0_v, pred1_v, p_v, accv,
                    sem_i, sem_e, sem_mt, sem0, sem1, sem2):
    cid = lax.axis_index("c")
    sid = lax.axis_index("s")
    wid = cid * _NSUB + sid
    base = wid * _PER
    b = lax.div(base, _K)

    cmi = pltpu.async_copy(ind.at[pl.ds(base, _PER)], ind_v, sem_i)
    cme = pltpu.async_copy(inde_flat.at[pl.ds(3 * base, 3 * _PER)], inde_v, sem_e)
    cmm = pltpu.async_copy(mask.at[pl.ds(base, _PER)], mask_v, sem_mt)
    cmt = pltpu.async_copy(tgt_flat.at[pl.ds(2 * base, 2 * _PER)], tgt_v, sem_mt)

    iota = lax.iota(jnp.int32, 16)
    cmi.wait()
    for c in range(_NCHUNK):
        sl = pl.ds(16 * c, 16)
        p0 = b * (_C * _HW) + ind_v[sl]
        idx0_v[sl] = p0
        idx1_v[sl] = p0 + _HW
    cp0 = pltpu.async_copy(out_flat.at[idx0_v], pred0_v, sem0)
    cp1 = pltpu.async_copy(out_flat.at[idx1_v], pred1_v, sem1)

    cme.wait()
    for c in range(_NCHUNK):
        sl = pl.ds(16 * c, 16)
        j3 = (iota + 16 * c) * 3
        i0 = plsc.load_gather(inde_v, [j3])
        i1 = plsc.load_gather(inde_v, [j3 + 1])
        i2 = plsc.load_gather(inde_v, [j3 + 2])
        idxh_v[sl] = b * (_NC * _HW) + i0 * _HW + i1 * _W + i2
    cph = pltpu.async_copy(hm_flat.at[idxh_v], p_v, sem2)

    cmm.wait()
    cmt.wait()
    cp0.wait()
    cp1.wait()
    cph.wait()

    acc = jnp.zeros((16,), jnp.float32)
    mac = jnp.zeros((16,), jnp.float32)
    for c in range(_NCHUNK):
        sl = pl.ds(16 * c, 16)
        j2 = (iota + 16 * c) * 2
        t0 = plsc.load_gather(tgt_v, [j2])
        t1 = plsc.load_gather(tgt_v, [j2 + 1])
        s0 = _smooth_l1_vec(pred0_v[sl] - t0)
        s1 = _smooth_l1_vec(pred1_v[sl] - t1)
        w = 1.0 + p_v[sl]
        m = mask_v[sl]
        acc = acc + (s0 + s1) * (w * w * m * 0.5)
        mac = mac + m
    accv[0, :] = acc
    accv[1, :] = mac
    pltpu.sync_copy(accv, o_parts.at[wid])


def _tc_finish_kernel(parts_ref, out_ref):
    a = parts_ref[...]
    num = jnp.sum(a[:, 0, :])
    den = jnp.sum(a[:, 1, :])
    out_ref[...] = jnp.broadcast_to(num / den, (1, 1))


@jax.jit
def kernel(output, mask, ind, target, inde, hm):
    out_flat = output.reshape(-1)
    hm_flat = hm.reshape(-1)
    ind_f = ind.reshape(-1).astype(jnp.int32)
    inde_flat = inde.reshape(-1).astype(jnp.int32)
    mask_f = mask.reshape(-1)
    tgt_flat = target.reshape(-1)

    f32 = jnp.float32
    i32 = jnp.int32
    sc_run = pl.kernel(
        _sc_loss_kernel,
        out_type=jax.ShapeDtypeStruct((_NW, 2, 16), f32),
        mesh=plsc.VectorSubcoreMesh(core_axis_name="c", subcore_axis_name="s"),
        compiler_params=pltpu.CompilerParams(needs_layout_passes=False),
        scratch_types=[
            pltpu.VMEM((_PER,), i32),
            pltpu.VMEM((3 * _PER,), i32),
            pltpu.VMEM((_PER,), f32),
            pltpu.VMEM((2 * _PER,), f32),
            pltpu.VMEM((_PER,), i32),
            pltpu.VMEM((_PER,), i32),
            pltpu.VMEM((_PER,), i32),
            pltpu.VMEM((_PER,), f32),
            pltpu.VMEM((_PER,), f32),
            pltpu.VMEM((_PER,), f32),
            pltpu.VMEM((2, 16), f32),
            pltpu.SemaphoreType.DMA,
            pltpu.SemaphoreType.DMA,
            pltpu.SemaphoreType.DMA,
            pltpu.SemaphoreType.DMA,
            pltpu.SemaphoreType.DMA,
            pltpu.SemaphoreType.DMA,
        ],
    )
    parts = sc_run(out_flat, hm_flat, ind_f, inde_flat, mask_f, tgt_flat)

    loss = pl.pallas_call(
        _tc_finish_kernel,
        out_shape=jax.ShapeDtypeStruct((1, 1), f32),
    )(parts)
    return loss[0, 0]


# speedup vs baseline: 1.0043x; 1.0043x over previous
import jax
import jax.numpy as jnp
from jax import lax
from jax.experimental import pallas as pl
from jax.experimental.pallas import tpu as pltpu
from jax.experimental.pallas import tpu_sc as plsc

_B, _C, _H, _W, _K, _NC = 8, 2, 128, 128, 128, 80
_HW = _H * _W
_NSLOT = _B * _K
_NCORE = 2
_NSUB = 16
_NW = _NCORE * _NSUB
_PER = _NSLOT // _NW
_NCHUNK = _PER // 16


def _smooth_l1_vec(d):
    a = jnp.abs(d)
    return jnp.where(a < 1.0, 0.5 * a * a, a - 0.5)


def _sc_loss_kernel(out_flat, hm_flat, ind, inde_flat, mask, tgt_flat,
                    o_parts,
                    ind_v, inde_v, mask_v, tgt_v,
                    idxp_v, idxh_v,
                    pred_v, p_v, accv,
                    sem_i, sem_e, sem_mt, sem0, sem2):
    cid = lax.axis_index("c")
    sid = lax.axis_index("s")
    wid = cid * _NSUB + sid
    base = wid * _PER
    b = lax.div(base, _K)

    cmi = pltpu.async_copy(ind.at[pl.ds(base, _PER)], ind_v, sem_i)
    cme = pltpu.async_copy(inde_flat.at[pl.ds(3 * base, 3 * _PER)], inde_v, sem_e)
    cmm = pltpu.async_copy(mask.at[pl.ds(base, _PER)], mask_v, sem_mt)
    cmt = pltpu.async_copy(tgt_flat.at[pl.ds(2 * base, 2 * _PER)], tgt_v, sem_mt)

    iota = lax.iota(jnp.int32, 16)
    cmi.wait()
    for c in range(_NCHUNK):
        sl = pl.ds(16 * c, 16)
        p0 = b * (_C * _HW) + ind_v[sl]
        j2 = (iota + 16 * c) * 2
        plsc.store_scatter(idxp_v, [j2], p0)
        plsc.store_scatter(idxp_v, [j2 + 1], p0 + _HW)
    cpp = pltpu.async_copy(out_flat.at[idxp_v], pred_v, sem0)

    cme.wait()
    for c in range(_NCHUNK):
        sl = pl.ds(16 * c, 16)
        j3 = (iota + 16 * c) * 3
        i0 = plsc.load_gather(inde_v, [j3])
        i1 = plsc.load_gather(inde_v, [j3 + 1])
        i2 = plsc.load_gather(inde_v, [j3 + 2])
        idxh_v[sl] = b * (_NC * _HW) + i0 * _HW + i1 * _W + i2
    cph = pltpu.async_copy(hm_flat.at[idxh_v], p_v, sem2)

    cmm.wait()
    cmt.wait()
    cpp.wait()
    cph.wait()

    acc = jnp.zeros((16,), jnp.float32)
    mac = jnp.zeros((16,), jnp.float32)
    for c in range(2 * _NCHUNK):
        sl = pl.ds(16 * c, 16)
        j = iota + 16 * c
        slot = lax.shift_right_logical(j, 1)
        s = _smooth_l1_vec(pred_v[sl] - tgt_v[sl])
        w = 1.0 + plsc.load_gather(p_v, [slot])
        m = plsc.load_gather(mask_v, [slot])
        acc = acc + s * (w * w * m * 0.5)
    for c in range(_NCHUNK):
        mac = mac + mask_v[pl.ds(16 * c, 16)]
    accv[0, :] = acc
    accv[1, :] = mac
    pltpu.sync_copy(accv, o_parts.at[wid])


def _tc_finish_kernel(parts_ref, out_ref):
    a = parts_ref[...]
    num = jnp.sum(a[:, 0, :])
    den = jnp.sum(a[:, 1, :])
    out_ref[...] = jnp.broadcast_to(num / den, (1, 1))


@jax.jit
def kernel(output, mask, ind, target, inde, hm):
    out_flat = output.reshape(-1)
    hm_flat = hm.reshape(-1)
    ind_f = ind.reshape(-1).astype(jnp.int32)
    inde_flat = inde.reshape(-1).astype(jnp.int32)
    mask_f = mask.reshape(-1)
    tgt_flat = target.reshape(-1)

    f32 = jnp.float32
    i32 = jnp.int32
    sc_run = pl.kernel(
        _sc_loss_kernel,
        out_type=jax.ShapeDtypeStruct((_NW, 2, 16), f32),
        mesh=plsc.VectorSubcoreMesh(core_axis_name="c", subcore_axis_name="s"),
        compiler_params=pltpu.CompilerParams(needs_layout_passes=False),
        scratch_types=[
            pltpu.VMEM((_PER,), i32),
            pltpu.VMEM((3 * _PER,), i32),
            pltpu.VMEM((_PER,), f32),
            pltpu.VMEM((2 * _PER,), f32),
            pltpu.VMEM((2 * _PER,), i32),
            pltpu.VMEM((_PER,), i32),
            pltpu.VMEM((2 * _PER,), f32),
            pltpu.VMEM((_PER,), f32),
            pltpu.VMEM((2, 16), f32),
            pltpu.SemaphoreType.DMA,
            pltpu.SemaphoreType.DMA,
            pltpu.SemaphoreType.DMA,
            pltpu.SemaphoreType.DMA,
            pltpu.SemaphoreType.DMA,
        ],
    )
    parts = sc_run(out_flat, hm_flat, ind_f, inde_flat, mask_f, tgt_flat)

    loss = pl.pallas_call(
        _tc_finish_kernel,
        out_shape=jax.ShapeDtypeStruct((1, 1), f32),
    )(parts)
    return loss[0, 0]
